# baseline (speedup 1.0000x reference)
import functools

import jax
import jax.numpy as jnp
from jax import lax
from jax.experimental import pallas as pl
from jax.experimental.pallas import tpu as pltpu
from jax.experimental.pallas import tpu_sc as plsc

B = 16
L = 4096
D = 8


def _build():
    mesh = plsc.VectorSubcoreMesh(core_axis_name="c", subcore_axis_name="s")

    @functools.partial(
        pl.kernel,
        mesh=mesh,
        out_type=jax.ShapeDtypeStruct((B * L * D,), jnp.float32),
        scratch_types=[
            pltpu.VMEM((16,), jnp.float32),
        ],
    )
    def k(lens_hbm, out_hbm, buf):
        c = lax.axis_index("c")
        s = lax.axis_index("s")
        wid = c * 16 + s
        buf[...] = jnp.full((16,), 1.0, jnp.float32)
        pltpu.sync_copy(buf, out_hbm.at[pl.ds(wid * 16, 16)])

    return k


_K = _build()


def kernel(time_seqs, seq_lengths, W, b):
    out_flat = _K(seq_lengths.astype(jnp.int32))
    return out_flat.reshape(B, L, D)


# speedup vs baseline: 2.5607x; 2.5607x over previous
import functools

import jax
import jax.numpy as jnp
from jax import lax
from jax.experimental import pallas as pl
from jax.experimental.pallas import tpu as pltpu
from jax.experimental.pallas import tpu_sc as plsc

B = 16
L = 4096
D = 8
HALF = L // 2
LANES = 16
LT_PER_HALF = HALF // 128
SECS_PER_DAY = 86400


def _take(v, idx):
    return lax.gather(
        v,
        idx[:, None],
        lax.GatherDimensionNumbers(
            offset_dims=(), collapsed_slice_dims=(0,), start_index_map=(0,)
        ),
        slice_sizes=(1,),
        mode=lax.GatherScatterMode.PROMISE_IN_BOUNDS,
    )


def _mod86400(chunk):
    t = lax.shift_right_logical(chunk, 7).astype(jnp.float32)
    low = (chunk & 127).astype(jnp.float32)
    q = (t * jnp.float32(1.0 / 675.0)).astype(jnp.int32).astype(jnp.float32)
    r = t - q * 675.0
    r = jnp.where(r < 0.0, r + 675.0, r)
    r = jnp.where(r >= 675.0, r - 675.0, r)
    return r * 128.0 + low


def _build():
    mesh = plsc.VectorSubcoreMesh(core_axis_name="c", subcore_axis_name="s")

    @functools.partial(
        pl.kernel,
        mesh=mesh,
        out_type=jax.ShapeDtypeStruct((B * L * D,), jnp.float32),
        scratch_types=[
            pltpu.VMEM((HALF,), jnp.int32),
            pltpu.VMEM((HALF,), jnp.float32),
            pltpu.VMEM((LANES,), jnp.int32),
            pltpu.VMEM((LANES,), jnp.float32),
            pltpu.VMEM((2 * LANES,), jnp.float32),
            pltpu.VMEM((2 * LANES,), jnp.float32),
            pltpu.VMEM_SHARED((LANES, 2 * LANES), jnp.float32),
            pltpu.VMEM((HALF * D,), jnp.float32),
            pltpu.SemaphoreType.DMA,
            pltpu.SemaphoreType.DMA,
        ],
    )
    def embed_kernel(
        ts_hbm, lens_hbm, w_hbm, b_hbm, out_hbm,
        ts_v, secs_v, lens_v, wb_v, pair_v, nbr_v, sh_x, out_v, isem, osem,
    ):
        c = lax.axis_index("c")
        s = lax.axis_index("s")
        wid = c * 16 + s
        row = wid // 2
        half = wid % 2
        rt = row // 8
        ir = row % 8

        tbase = rt * 32 + half * LT_PER_HALF
        in_cps = [
            pltpu.async_copy(
                ts_hbm.at[tbase + lt, ir], ts_v.at[pl.ds(lt * 128, 128)], isem
            )
            for lt in range(LT_PER_HALF)
        ]
        lens_cp = pltpu.async_copy(lens_hbm, lens_v, isem)
        w_cp = pltpu.async_copy(w_hbm, wb_v.at[pl.ds(0, D)], isem)
        b_cp = pltpu.async_copy(b_hbm, wb_v.at[pl.ds(D, D)], isem)
        for cp in in_cps:
            cp.wait()

        iota = lax.iota(jnp.int32, LANES)

        def p1_body(i, carry):
            mn0, mx0, mn1, mx1 = carry
            base = i * 128
            for ch in range(8):
                secs = _mod86400(ts_v[pl.ds(base + ch * LANES, LANES)])
                secs_v[pl.ds(base + ch * LANES, LANES)] = secs
                if ch % 2 == 0:
                    mn0 = jnp.minimum(mn0, secs)
                    mx0 = jnp.maximum(mx0, secs)
                else:
                    mn1 = jnp.minimum(mn1, secs)
                    mx1 = jnp.maximum(mx1, secs)
            return mn0, mx0, mn1, mx1

        big = jnp.full((LANES,), 3.0e38, jnp.float32)
        mn0, mx0, mn1, mx1 = lax.fori_loop(
            0, LT_PER_HALF, p1_body, (big, -big, big, -big)
        )
        mn_v = jnp.minimum(mn0, mn1)
        mx_v = jnp.maximum(mx0, mx1)

        pair_v[pl.ds(0, LANES)] = mn_v
        pair_v[pl.ds(LANES, LANES)] = mx_v
        pltpu.sync_copy(pair_v, sh_x.at[s])
        plsc.subcore_barrier()
        pltpu.sync_copy(sh_x.at[s ^ 1], nbr_v)
        lens_cp.wait()
        w_cp.wait()
        b_cp.wait()
        mn_v = jnp.minimum(mn_v, nbr_v[pl.ds(0, LANES)])
        mx_v = jnp.maximum(mx_v, nbr_v[pl.ds(LANES, LANES)])

        for step in (8, 4, 2, 1):
            perm = iota ^ step
            mn_v = jnp.minimum(mn_v, _take(mn_v, perm))
            mx_v = jnp.maximum(mx_v, _take(mx_v, perm))
        inv_b = 1.0 / (mx_v - mn_v)

        len_b = _take(lens_v[...], jnp.broadcast_to(row, (LANES,)).astype(jnp.int32))
        wb_vec = wb_v[...]
        w_s = [_take(wb_vec, jnp.full((LANES,), d, jnp.int32)) for d in range(D)]
        b_s = [_take(wb_vec, jnp.full((LANES,), D + d, jnp.int32)) for d in range(D)]

        rbase = half * HALF

        def p2_body(i, carry):
            base = i * 128
            obase = i * (D * 128)
            nms, vfs = [], []
            for ch in range(8):
                s16 = secs_v[pl.ds(base + ch * LANES, LANES)]
                n = (s16 - mn_v) * inv_b
                valid = ((rbase + base + ch * LANES) + iota) < len_b
                nms.append(jnp.where(valid, n, 0.0))
                vfs.append(jnp.where(valid, 1.0, 0.0))
            for ch in range(8):
                for d in range(D):
                    out_v[pl.ds(obase + d * 128 + ch * LANES, LANES)] = (
                        nms[ch] * w_s[d] + vfs[ch] * b_s[d]
                    )
            return carry

        obase_hbm = wid * HALF * D
        hwords = (HALF * D) // 2
        out_cps = []
        for g in range(2):
            lax.fori_loop(
                g * (LT_PER_HALF // 2), (g + 1) * (LT_PER_HALF // 2), p2_body, 0
            )
            out_cps.append(
                pltpu.async_copy(
                    out_v.at[pl.ds(g * hwords, hwords)],
                    out_hbm.at[pl.ds(obase_hbm + g * hwords, hwords)],
                    osem,
                )
            )
        for cp in out_cps:
            cp.wait()

    return embed_kernel


_EMBED = _build()


def kernel(time_seqs, seq_lengths, W, b):
    ts_t = (
        time_seqs.astype(jnp.int32)
        .reshape(2, 8, 32, 128)
        .transpose(0, 2, 1, 3)
        .reshape(64, 8, 128)
    )
    lens = seq_lengths.astype(jnp.int32)
    out_flat = _EMBED(ts_t, lens, W.reshape(D), b)
    return (
        out_flat.reshape(B, 32, D, 128)
        .transpose(0, 1, 3, 2)
        .reshape(B, L, D)
    )
